# baseline (device time: 1292653 ns/iter reference)
import jax
import jax.numpy as jnp
from jax import lax
from jax.experimental import pallas as pl
from jax.experimental.pallas import tpu as pltpu

N_DEV = 32
B = 2048
D = 1024
CH = B // N_DEV
C2 = CH // 2
HALF = B // 2
DH = D // 2
NSLOT = 4
HOPS = N_DEV - 1


def _neighbor_barrier(left, right):
    barrier = pltpu.get_barrier_semaphore()
    for nbr in (left, right):
        pl.semaphore_signal(
            barrier, inc=1, device_id=(nbr,),
            device_id_type=pl.DeviceIdType.MESH,
        )
    pl.semaphore_wait(barrier, 2)


def _rdma(src, dst, send_sem, recv_sem, dev):
    return pltpu.make_async_remote_copy(
        src_ref=src, dst_ref=dst, send_sem=send_sem, recv_sem=recv_sem,
        device_id=(dev,), device_id_type=pl.DeviceIdType.MESH,
    )


def _ag_body(x_ref, out_ref, ct, cb, t_s, t_r, b_s, b_r):
    my = lax.axis_index("i")
    left = (my - 1) % N_DEV
    right = (my + 1) % N_DEV
    _neighbor_barrier(left, right)

    out_ref[pl.ds(my * CH, CH), :] = x_ref[...]
    ct[0, :, :] = x_ref[:, 0:DH]
    cb[0, :, :] = x_ref[:, DH:D]

    pending = []
    for h in range(HOPS):
        ss, rr = h % NSLOT, (h + 1) % NSLOT
        rt = _rdma(ct.at[ss], ct.at[rr], t_s.at[h], t_r.at[h], right)
        rb = _rdma(cb.at[ss], cb.at[rr], b_s.at[h], b_r.at[h], left)
        rt.start()
        rb.start()
        pending += [rt, rb]
        rt.wait_recv()
        rb.wait_recv()
        to = (my - h - 1) % N_DEV
        bo = (my + h + 1) % N_DEV
        out_ref[pl.ds(to * CH, CH), 0:DH] = ct[rr]
        out_ref[pl.ds(bo * CH, CH), DH:D] = cb[rr]
    for r in pending:
        r.wait_send()


def _allgather(x):
    return pl.pallas_call(
        _ag_body,
        out_shape=jax.ShapeDtypeStruct((B, D), x.dtype),
        in_specs=[pl.BlockSpec(memory_space=pltpu.VMEM)],
        out_specs=pl.BlockSpec(memory_space=pltpu.VMEM),
        scratch_shapes=[
            pltpu.VMEM((NSLOT, CH, DH), x.dtype),
            pltpu.VMEM((NSLOT, CH, DH), x.dtype),
            pltpu.SemaphoreType.DMA((HOPS,)),
            pltpu.SemaphoreType.DMA((HOPS,)),
            pltpu.SemaphoreType.DMA((HOPS,)),
            pltpu.SemaphoreType.DMA((HOPS,)),
        ],
        compiler_params=pltpu.CompilerParams(collective_id=0),
    )(x)


_RB = 256


def _layer_body(x_ref, win_ref, wout_ref, out_ref):
    h = jnp.maximum(
        jnp.dot(x_ref[...], win_ref[...], preferred_element_type=jnp.float32),
        0.0,
    )
    out_ref[...] = jnp.dot(h, wout_ref[...], preferred_element_type=jnp.float32)


def _layer(x_full, win, wout):
    hdim = win.shape[1]
    return pl.pallas_call(
        _layer_body,
        grid=(B // _RB,),
        in_specs=[
            pl.BlockSpec((_RB, D), lambda r: (r, 0)),
            pl.BlockSpec((D, hdim), lambda r: (0, 0)),
            pl.BlockSpec((hdim, D), lambda r: (0, 0)),
        ],
        out_specs=pl.BlockSpec((_RB, D), lambda r: (r, 0)),
        out_shape=jax.ShapeDtypeStruct((B, D), jnp.float32),
    )(x_full, win, wout)


def _ar_body(p_ref, out_ref, ct, cb,
             t_rs_s, t_rs_r, t_ag_s, t_ag_r,
             b_rs_s, b_rs_r, b_ag_s, b_ag_r):
    my = lax.axis_index("i")
    left = (my - 1) % N_DEV
    right = (my + 1) % N_DEV
    _neighbor_barrier(left, right)

    def toprows(c):
        return pl.ds(c * C2, C2)

    def botrows(c):
        return pl.ds(HALF + c * C2, C2)

    pending = []

    ct[0, :, :] = p_ref[toprows(my), :]
    cb[0, :, :] = p_ref[botrows(my), :]
    for h in range(HOPS):
        ss, rr = h % NSLOT, (h + 1) % NSLOT
        rt = _rdma(ct.at[ss], ct.at[rr], t_rs_s.at[h], t_rs_r.at[h], right)
        rb = _rdma(cb.at[ss], cb.at[rr], b_rs_s.at[h], b_rs_r.at[h], left)
        rt.start()
        rb.start()
        pending += [rt, rb]
        rt.wait_recv()
        rb.wait_recv()
        tc = (my - h - 1) % N_DEV
        bc = (my + h + 1) % N_DEV
        ct[rr, :, :] = ct[rr] + p_ref[toprows(tc), :]
        cb[rr, :, :] = cb[rr] + p_ref[botrows(bc), :]

    t_own = (my + 1) % N_DEV
    b_own = (my - 1) % N_DEV
    last = HOPS % NSLOT
    out_ref[toprows(t_own), :] = ct[last]
    out_ref[botrows(b_own), :] = cb[last]

    for h in range(HOPS):
        ss, rr = (last + h) % NSLOT, (last + h + 1) % NSLOT
        rt = _rdma(ct.at[ss], ct.at[rr], t_ag_s.at[h], t_ag_r.at[h], right)
        rb = _rdma(cb.at[ss], cb.at[rr], b_ag_s.at[h], b_ag_r.at[h], left)
        rt.start()
        rb.start()
        pending += [rt, rb]
        rt.wait_recv()
        rb.wait_recv()
        to = (my - h) % N_DEV
        bo = (my + h) % N_DEV
        out_ref[toprows(to), :] = ct[rr]
        out_ref[botrows(bo), :] = cb[rr]

    for r in pending:
        r.wait_send()


def _allreduce(p):
    return pl.pallas_call(
        _ar_body,
        out_shape=jax.ShapeDtypeStruct((B, D), p.dtype),
        in_specs=[pl.BlockSpec(memory_space=pltpu.VMEM)],
        out_specs=pl.BlockSpec(memory_space=pltpu.VMEM),
        scratch_shapes=[
            pltpu.VMEM((NSLOT, C2, D), p.dtype),
            pltpu.VMEM((NSLOT, C2, D), p.dtype),
        ] + [pltpu.SemaphoreType.DMA((HOPS,)) for _ in range(8)],
        compiler_params=pltpu.CompilerParams(collective_id=1),
    )(p)


def kernel(x, Win0, Wout0, Win1, Wout1, Win2, Wout2):
    x_full = _allgather(x)
    for win, wout in ((Win0, Wout0), (Win1, Wout1), (Win2, Wout2)):
        partial = _layer(x_full, win, wout)
        x_full = _allreduce(partial)
    return x_full


# device time: 788946 ns/iter; 1.6385x vs baseline; 1.6385x over previous
import numpy as np

import jax
import jax.numpy as jnp
from jax import lax
from jax.experimental import pallas as pl
from jax.experimental.pallas import tpu as pltpu

N_DEV = 32
B = 2048
D = 1024
CH = B // N_DEV
C2 = CH // 2
HALF = B // 2
DH = D // 2
NSLOT = 4
HOPS = N_DEV - 1
SERIALIZED = True


def _build_cycle():
    logical_coords = []
    for z in range(4):
        for y in range(4):
            row = [(0, y, z), (1, y, z)] if y % 2 == 0 else [(1, y, z), (0, y, z)]
            logical_coords.extend(row)
    lid = {c: i for i, c in enumerate(logical_coords)}
    yz = []
    for z in range(4):
        ys = range(4) if z % 2 == 0 else range(3, -1, -1)
        yz.extend((y, z) for y in ys)
    cycle = [(0, y, z) for (y, z) in yz] + [(1, y, z) for (y, z) in reversed(yz)]
    cyc_lid = [lid[c] for c in cycle]
    pos = np.argsort(cyc_lid).astype(np.int32)
    nxt = np.empty(N_DEV, np.int32)
    prv = np.empty(N_DEV, np.int32)
    for p, l in enumerate(cyc_lid):
        nxt[l] = cyc_lid[(p + 1) % N_DEV]
        prv[l] = cyc_lid[(p - 1) % N_DEV]
    return np.array(cyc_lid, np.int32), pos, nxt, prv


_CYC_LID, _POS, _NXT, _PRV = _build_cycle()


def _ring_meta():
    my = lax.axis_index("i")
    pos = jnp.asarray(_POS)[my]
    nxt = jnp.asarray(_NXT)[my]
    prv = jnp.asarray(_PRV)[my]
    return jnp.stack([pos, nxt, prv, my]).astype(jnp.int32)


def _neighbor_barrier(nxt, prv):
    barrier = pltpu.get_barrier_semaphore()
    for nbr in (nxt, prv):
        pl.semaphore_signal(
            barrier, inc=1, device_id=(nbr,),
            device_id_type=pl.DeviceIdType.MESH,
        )
    pl.semaphore_wait(barrier, 2)


def _rdma(src, dst, send_sem, recv_sem, dev):
    return pltpu.make_async_remote_copy(
        src_ref=src, dst_ref=dst, send_sem=send_sem, recv_sem=recv_sem,
        device_id=(dev,), device_id_type=pl.DeviceIdType.MESH,
    )


def _step_waits(rt, rb, hist, h):
    if SERIALIZED:
        rt.wait()
        rb.wait()
    else:
        if h >= 2:
            hist[h - 2][0].wait_send()
            hist[h - 2][1].wait_send()
        hist.append((rt, rb))
        rt.wait_recv()
        rb.wait_recv()


def _drain(hist):
    if not SERIALIZED:
        for pair in hist[-2:]:
            pair[0].wait_send()
            pair[1].wait_send()


def _ag_body(meta_ref, tbl_ref, x_ref, out_ref, ct, cb, t_s, t_r, b_s, b_r):
    pos, nxt, prv, my = (meta_ref[i] for i in range(4))
    _neighbor_barrier(nxt, prv)

    out_ref[pl.ds(my * CH, CH), :] = x_ref[...]
    ct[0, :, :] = x_ref[:, 0:DH]
    cb[0, :, :] = x_ref[:, DH:D]

    hist = []
    for h in range(HOPS):
        ss, rr = h % NSLOT, (h + 1) % NSLOT
        rt = _rdma(ct.at[ss], ct.at[rr], t_s.at[h], t_r.at[h], nxt)
        rb = _rdma(cb.at[ss], cb.at[rr], b_s.at[h], b_r.at[h], prv)
        rt.start()
        rb.start()
        _step_waits(rt, rb, hist, h)
        to = tbl_ref[(pos - h - 1) % N_DEV]
        bo = tbl_ref[(pos + h + 1) % N_DEV]
        out_ref[pl.ds(to * CH, CH), 0:DH] = ct[rr]
        out_ref[pl.ds(bo * CH, CH), DH:D] = cb[rr]
    _drain(hist)


def _allgather(x, meta, tbl):
    return pl.pallas_call(
        _ag_body,
        out_shape=jax.ShapeDtypeStruct((B, D), x.dtype),
        in_specs=[
            pl.BlockSpec(memory_space=pltpu.SMEM),
            pl.BlockSpec(memory_space=pltpu.SMEM),
            pl.BlockSpec(memory_space=pltpu.VMEM),
        ],
        out_specs=pl.BlockSpec(memory_space=pltpu.VMEM),
        scratch_shapes=[
            pltpu.VMEM((NSLOT, CH, DH), x.dtype),
            pltpu.VMEM((NSLOT, CH, DH), x.dtype),
            pltpu.SemaphoreType.DMA((HOPS,)),
            pltpu.SemaphoreType.DMA((HOPS,)),
            pltpu.SemaphoreType.DMA((HOPS,)),
            pltpu.SemaphoreType.DMA((HOPS,)),
        ],
        compiler_params=pltpu.CompilerParams(collective_id=0),
    )(meta, tbl, x)


_RB = 256


def _layer_body(x_ref, win_ref, wout_ref, out_ref):
    h = jnp.maximum(
        jnp.dot(x_ref[...], win_ref[...], preferred_element_type=jnp.float32),
        0.0,
    )
    out_ref[...] = jnp.dot(h, wout_ref[...], preferred_element_type=jnp.float32)


def _layer(x_full, win, wout):
    hdim = win.shape[1]
    return pl.pallas_call(
        _layer_body,
        grid=(B // _RB,),
        in_specs=[
            pl.BlockSpec((_RB, D), lambda r: (r, 0)),
            pl.BlockSpec((D, hdim), lambda r: (0, 0)),
            pl.BlockSpec((hdim, D), lambda r: (0, 0)),
        ],
        out_specs=pl.BlockSpec((_RB, D), lambda r: (r, 0)),
        out_shape=jax.ShapeDtypeStruct((B, D), jnp.float32),
    )(x_full, win, wout)


def _ar_body(meta_ref, p_ref, out_ref, ct, cb,
             t_rs_s, t_rs_r, t_ag_s, t_ag_r,
             b_rs_s, b_rs_r, b_ag_s, b_ag_r):
    pos, nxt, prv, _ = (meta_ref[i] for i in range(4))
    _neighbor_barrier(nxt, prv)

    def toprows(c):
        return pl.ds(c * C2, C2)

    def botrows(c):
        return pl.ds(HALF + c * C2, C2)

    ct[0, :, :] = p_ref[toprows(pos), :]
    cb[0, :, :] = p_ref[botrows(pos), :]
    hist = []
    for h in range(HOPS):
        ss, rr = h % NSLOT, (h + 1) % NSLOT
        rt = _rdma(ct.at[ss], ct.at[rr], t_rs_s.at[h], t_rs_r.at[h], nxt)
        rb = _rdma(cb.at[ss], cb.at[rr], b_rs_s.at[h], b_rs_r.at[h], prv)
        rt.start()
        rb.start()
        _step_waits(rt, rb, hist, h)
        tc = (pos - h - 1) % N_DEV
        bc = (pos + h + 1) % N_DEV
        ct[rr, :, :] = ct[rr] + p_ref[toprows(tc), :]
        cb[rr, :, :] = cb[rr] + p_ref[botrows(bc), :]
    _drain(hist)

    last = HOPS % NSLOT
    out_ref[toprows((pos + 1) % N_DEV), :] = ct[last]
    out_ref[botrows((pos - 1) % N_DEV), :] = cb[last]

    hist = []
    for h in range(HOPS):
        ss, rr = (last + h) % NSLOT, (last + h + 1) % NSLOT
        rt = _rdma(ct.at[ss], ct.at[rr], t_ag_s.at[h], t_ag_r.at[h], nxt)
        rb = _rdma(cb.at[ss], cb.at[rr], b_ag_s.at[h], b_ag_r.at[h], prv)
        rt.start()
        rb.start()
        _step_waits(rt, rb, hist, h)
        out_ref[toprows((pos - h) % N_DEV), :] = ct[rr]
        out_ref[botrows((pos + h) % N_DEV), :] = cb[rr]
    _drain(hist)


def _allreduce(p, meta):
    return pl.pallas_call(
        _ar_body,
        out_shape=jax.ShapeDtypeStruct((B, D), p.dtype),
        in_specs=[
            pl.BlockSpec(memory_space=pltpu.SMEM),
            pl.BlockSpec(memory_space=pltpu.VMEM),
        ],
        out_specs=pl.BlockSpec(memory_space=pltpu.VMEM),
        scratch_shapes=[
            pltpu.VMEM((NSLOT, C2, D), p.dtype),
            pltpu.VMEM((NSLOT, C2, D), p.dtype),
        ] + [pltpu.SemaphoreType.DMA((HOPS,)) for _ in range(8)],
        compiler_params=pltpu.CompilerParams(collective_id=1),
    )(meta, p)


def kernel(x, Win0, Wout0, Win1, Wout1, Win2, Wout2):
    meta = _ring_meta()
    tbl = jnp.asarray(_CYC_LID)
    x_full = _allgather(x, meta, tbl)
    for win, wout in ((Win0, Wout0), (Win1, Wout1), (Win2, Wout2)):
        partial = _layer(x_full, win, wout)
        x_full = _allreduce(partial, meta)
    return x_full


# device time: 788743 ns/iter; 1.6389x vs baseline; 1.0003x over previous
import numpy as np

import jax
import jax.numpy as jnp
from jax import lax
from jax.experimental import pallas as pl
from jax.experimental.pallas import tpu as pltpu

N_DEV = 32
B = 2048
D = 1024
CH = B // N_DEV
C2 = CH // 2
HALF = B // 2
DH = D // 2
NSLOT = 4
HOPS = N_DEV - 1
SERIALIZED = False


def _build_cycle():
    logical_coords = []
    for z in range(4):
        for y in range(4):
            row = [(0, y, z), (1, y, z)] if y % 2 == 0 else [(1, y, z), (0, y, z)]
            logical_coords.extend(row)
    lid = {c: i for i, c in enumerate(logical_coords)}
    yz = []
    for z in range(4):
        ys = range(4) if z % 2 == 0 else range(3, -1, -1)
        yz.extend((y, z) for y in ys)
    cycle = [(0, y, z) for (y, z) in yz] + [(1, y, z) for (y, z) in reversed(yz)]
    cyc_lid = [lid[c] for c in cycle]
    pos = np.argsort(cyc_lid).astype(np.int32)
    nxt = np.empty(N_DEV, np.int32)
    prv = np.empty(N_DEV, np.int32)
    for p, l in enumerate(cyc_lid):
        nxt[l] = cyc_lid[(p + 1) % N_DEV]
        prv[l] = cyc_lid[(p - 1) % N_DEV]
    return np.array(cyc_lid, np.int32), pos, nxt, prv


_CYC_LID, _POS, _NXT, _PRV = _build_cycle()


def _ring_meta():
    my = lax.axis_index("i")
    pos = jnp.asarray(_POS)[my]
    nxt = jnp.asarray(_NXT)[my]
    prv = jnp.asarray(_PRV)[my]
    return jnp.stack([pos, nxt, prv, my]).astype(jnp.int32)


def _neighbor_barrier(nxt, prv):
    barrier = pltpu.get_barrier_semaphore()
    for nbr in (nxt, prv):
        pl.semaphore_signal(
            barrier, inc=1, device_id=(nbr,),
            device_id_type=pl.DeviceIdType.MESH,
        )
    pl.semaphore_wait(barrier, 2)


def _rdma(src, dst, send_sem, recv_sem, dev):
    return pltpu.make_async_remote_copy(
        src_ref=src, dst_ref=dst, send_sem=send_sem, recv_sem=recv_sem,
        device_id=(dev,), device_id_type=pl.DeviceIdType.MESH,
    )


def _step_waits(rt, rb, hist, h):
    if SERIALIZED:
        rt.wait()
        rb.wait()
    else:
        if h >= 2:
            hist[h - 2][0].wait_send()
            hist[h - 2][1].wait_send()
        hist.append((rt, rb))
        rt.wait_recv()
        rb.wait_recv()


def _drain(hist):
    if not SERIALIZED:
        for pair in hist[-2:]:
            pair[0].wait_send()
            pair[1].wait_send()


def _ag_body(meta_ref, tbl_ref, x_ref, out_ref, ct, cb, t_s, t_r, b_s, b_r):
    pos, nxt, prv, my = (meta_ref[i] for i in range(4))
    _neighbor_barrier(nxt, prv)

    out_ref[pl.ds(my * CH, CH), :] = x_ref[...]
    ct[0, :, :] = x_ref[:, 0:DH]
    cb[0, :, :] = x_ref[:, DH:D]

    hist = []
    for h in range(HOPS):
        ss, rr = h % NSLOT, (h + 1) % NSLOT
        rt = _rdma(ct.at[ss], ct.at[rr], t_s.at[h], t_r.at[h], nxt)
        rb = _rdma(cb.at[ss], cb.at[rr], b_s.at[h], b_r.at[h], prv)
        rt.start()
        rb.start()
        _step_waits(rt, rb, hist, h)
        to = tbl_ref[(pos - h - 1) % N_DEV]
        bo = tbl_ref[(pos + h + 1) % N_DEV]
        out_ref[pl.ds(to * CH, CH), 0:DH] = ct[rr]
        out_ref[pl.ds(bo * CH, CH), DH:D] = cb[rr]
    _drain(hist)


def _allgather(x, meta, tbl):
    return pl.pallas_call(
        _ag_body,
        out_shape=jax.ShapeDtypeStruct((B, D), x.dtype),
        in_specs=[
            pl.BlockSpec(memory_space=pltpu.SMEM),
            pl.BlockSpec(memory_space=pltpu.SMEM),
            pl.BlockSpec(memory_space=pltpu.VMEM),
        ],
        out_specs=pl.BlockSpec(memory_space=pltpu.VMEM),
        scratch_shapes=[
            pltpu.VMEM((NSLOT, CH, DH), x.dtype),
            pltpu.VMEM((NSLOT, CH, DH), x.dtype),
            pltpu.SemaphoreType.DMA((HOPS,)),
            pltpu.SemaphoreType.DMA((HOPS,)),
            pltpu.SemaphoreType.DMA((HOPS,)),
            pltpu.SemaphoreType.DMA((HOPS,)),
        ],
        compiler_params=pltpu.CompilerParams(collective_id=0),
    )(meta, tbl, x)


_RB = 256


def _layer_body(x_ref, win_ref, wout_ref, out_ref):
    h = jnp.maximum(
        jnp.dot(x_ref[...], win_ref[...], preferred_element_type=jnp.float32),
        0.0,
    )
    out_ref[...] = jnp.dot(h, wout_ref[...], preferred_element_type=jnp.float32)


def _layer(x_full, win, wout):
    hdim = win.shape[1]
    return pl.pallas_call(
        _layer_body,
        grid=(B // _RB,),
        in_specs=[
            pl.BlockSpec((_RB, D), lambda r: (r, 0)),
            pl.BlockSpec((D, hdim), lambda r: (0, 0)),
            pl.BlockSpec((hdim, D), lambda r: (0, 0)),
        ],
        out_specs=pl.BlockSpec((_RB, D), lambda r: (r, 0)),
        out_shape=jax.ShapeDtypeStruct((B, D), jnp.float32),
    )(x_full, win, wout)


def _ar_body(meta_ref, p_ref, out_ref, ct, cb,
             t_rs_s, t_rs_r, t_ag_s, t_ag_r,
             b_rs_s, b_rs_r, b_ag_s, b_ag_r):
    pos, nxt, prv, _ = (meta_ref[i] for i in range(4))
    _neighbor_barrier(nxt, prv)

    def toprows(c):
        return pl.ds(c * C2, C2)

    def botrows(c):
        return pl.ds(HALF + c * C2, C2)

    ct[0, :, :] = p_ref[toprows(pos), :]
    cb[0, :, :] = p_ref[botrows(pos), :]
    hist = []
    for h in range(HOPS):
        ss, rr = h % NSLOT, (h + 1) % NSLOT
        rt = _rdma(ct.at[ss], ct.at[rr], t_rs_s.at[h], t_rs_r.at[h], nxt)
        rb = _rdma(cb.at[ss], cb.at[rr], b_rs_s.at[h], b_rs_r.at[h], prv)
        rt.start()
        rb.start()
        _step_waits(rt, rb, hist, h)
        tc = (pos - h - 1) % N_DEV
        bc = (pos + h + 1) % N_DEV
        ct[rr, :, :] = ct[rr] + p_ref[toprows(tc), :]
        cb[rr, :, :] = cb[rr] + p_ref[botrows(bc), :]
    _drain(hist)

    last = HOPS % NSLOT
    out_ref[toprows((pos + 1) % N_DEV), :] = ct[last]
    out_ref[botrows((pos - 1) % N_DEV), :] = cb[last]

    hist = []
    for h in range(HOPS):
        ss, rr = (last + h) % NSLOT, (last + h + 1) % NSLOT
        rt = _rdma(ct.at[ss], ct.at[rr], t_ag_s.at[h], t_ag_r.at[h], nxt)
        rb = _rdma(cb.at[ss], cb.at[rr], b_ag_s.at[h], b_ag_r.at[h], prv)
        rt.start()
        rb.start()
        _step_waits(rt, rb, hist, h)
        out_ref[toprows((pos - h) % N_DEV), :] = ct[rr]
        out_ref[botrows((pos + h) % N_DEV), :] = cb[rr]
    _drain(hist)


def _allreduce(p, meta):
    return pl.pallas_call(
        _ar_body,
        out_shape=jax.ShapeDtypeStruct((B, D), p.dtype),
        in_specs=[
            pl.BlockSpec(memory_space=pltpu.SMEM),
            pl.BlockSpec(memory_space=pltpu.VMEM),
        ],
        out_specs=pl.BlockSpec(memory_space=pltpu.VMEM),
        scratch_shapes=[
            pltpu.VMEM((NSLOT, C2, D), p.dtype),
            pltpu.VMEM((NSLOT, C2, D), p.dtype),
        ] + [pltpu.SemaphoreType.DMA((HOPS,)) for _ in range(8)],
        compiler_params=pltpu.CompilerParams(collective_id=1),
    )(meta, p)


def kernel(x, Win0, Wout0, Win1, Wout1, Win2, Wout2):
    meta = _ring_meta()
    tbl = jnp.asarray(_CYC_LID)
    x_full = _allgather(x, meta, tbl)
    for win, wout in ((Win0, Wout0), (Win1, Wout1), (Win2, Wout2)):
        partial = _layer(x_full, win, wout)
        x_full = _allreduce(partial, meta)
    return x_full


# device time: 782875 ns/iter; 1.6512x vs baseline; 1.0075x over previous
import numpy as np

import jax
import jax.numpy as jnp
from jax import lax
from jax.experimental import pallas as pl
from jax.experimental.pallas import tpu as pltpu

N_DEV = 32
B = 2048
D = 1024
CH = B // N_DEV
C2 = CH // 2
HALF = B // 2
DH = D // 2
NSLOT = 4
HOPS = N_DEV - 1
SERIALIZED = False


def _build_cycle():
    logical_coords = []
    for z in range(4):
        for y in range(4):
            row = [(0, y, z), (1, y, z)] if y % 2 == 0 else [(1, y, z), (0, y, z)]
            logical_coords.extend(row)
    lid = {c: i for i, c in enumerate(logical_coords)}
    yz = []
    for z in range(4):
        ys = range(4) if z % 2 == 0 else range(3, -1, -1)
        yz.extend((y, z) for y in ys)
    cycle = [(0, y, z) for (y, z) in yz] + [(1, y, z) for (y, z) in reversed(yz)]
    cyc_lid = [lid[c] for c in cycle]
    pos = np.argsort(cyc_lid).astype(np.int32)
    nxt = np.empty(N_DEV, np.int32)
    prv = np.empty(N_DEV, np.int32)
    for p, l in enumerate(cyc_lid):
        nxt[l] = cyc_lid[(p + 1) % N_DEV]
        prv[l] = cyc_lid[(p - 1) % N_DEV]
    return np.array(cyc_lid, np.int32), pos, nxt, prv


_CYC_LID, _POS, _NXT, _PRV = _build_cycle()


def _ring_meta():
    my = lax.axis_index("i")
    pos = jnp.asarray(_POS)[my]
    nxt = jnp.asarray(_NXT)[my]
    prv = jnp.asarray(_PRV)[my]
    return jnp.stack([pos, nxt, prv, my]).astype(jnp.int32)


def _neighbor_barrier(nxt, prv):
    barrier = pltpu.get_barrier_semaphore()
    for nbr in (nxt, prv):
        pl.semaphore_signal(
            barrier, inc=1, device_id=(nbr,),
            device_id_type=pl.DeviceIdType.MESH,
        )
    pl.semaphore_wait(barrier, 2)


def _rdma(src, dst, send_sem, recv_sem, dev):
    return pltpu.make_async_remote_copy(
        src_ref=src, dst_ref=dst, send_sem=send_sem, recv_sem=recv_sem,
        device_id=(dev,), device_id_type=pl.DeviceIdType.MESH,
    )


def _dual_ring(ct, cb, t_s, t_r, b_s, b_r, nxt, prv, s0, on_t, on_b,
               mutates=False):
    if SERIALIZED:
        for h in range(HOPS):
            ss, rr = (s0 + h) % NSLOT, (s0 + h + 1) % NSLOT
            rt = _rdma(ct.at[ss], ct.at[rr], t_s.at[h], t_r.at[h], nxt)
            rb = _rdma(cb.at[ss], cb.at[rr], b_s.at[h], b_r.at[h], prv)
            rt.start()
            rb.start()
            rt.wait()
            rb.wait()
            on_t(h, rr)
            on_b(h, rr)
        return
    rts, rbs = [], []
    for h in range(HOPS):
        ss, rr = (s0 + h) % NSLOT, (s0 + h + 1) % NSLOT
        rts.append(_rdma(ct.at[ss], ct.at[rr], t_s.at[h], t_r.at[h], nxt))
        rbs.append(_rdma(cb.at[ss], cb.at[rr], b_s.at[h], b_r.at[h], prv))
    rts[0].start()
    rbs[0].start()
    for h in range(HOPS):
        rr = (s0 + h + 1) % NSLOT
        rts[h].wait_recv()
        if mutates:
            on_t(h, rr)
        if h + 1 < HOPS:
            rts[h + 1].start()
        rbs[h].wait_recv()
        if mutates:
            on_b(h, rr)
        if h + 1 < HOPS:
            rbs[h + 1].start()
        if not mutates:
            on_t(h, rr)
            on_b(h, rr)
        if h >= 2:
            rts[h - 2].wait_send()
            rbs[h - 2].wait_send()
    for r in rts[-2:] + rbs[-2:]:
        r.wait_send()


def _ag_body(meta_ref, tbl_ref, x_ref, out_ref, ct, cb, t_s, t_r, b_s, b_r):
    pos, nxt, prv, my = (meta_ref[i] for i in range(4))
    _neighbor_barrier(nxt, prv)

    out_ref[pl.ds(my * CH, CH), :] = x_ref[...]
    ct[0, :, :] = x_ref[:, 0:DH]
    cb[0, :, :] = x_ref[:, DH:D]

    def store_t(h, rr):
        to = tbl_ref[(pos - h - 1) % N_DEV]
        out_ref[pl.ds(to * CH, CH), 0:DH] = ct[rr]

    def store_b(h, rr):
        bo = tbl_ref[(pos + h + 1) % N_DEV]
        out_ref[pl.ds(bo * CH, CH), DH:D] = cb[rr]

    _dual_ring(ct, cb, t_s, t_r, b_s, b_r, nxt, prv, 0, store_t, store_b)


def _allgather(x, meta, tbl):
    return pl.pallas_call(
        _ag_body,
        out_shape=jax.ShapeDtypeStruct((B, D), x.dtype),
        in_specs=[
            pl.BlockSpec(memory_space=pltpu.SMEM),
            pl.BlockSpec(memory_space=pltpu.SMEM),
            pl.BlockSpec(memory_space=pltpu.VMEM),
        ],
        out_specs=pl.BlockSpec(memory_space=pltpu.VMEM),
        scratch_shapes=[
            pltpu.VMEM((NSLOT, CH, DH), x.dtype),
            pltpu.VMEM((NSLOT, CH, DH), x.dtype),
            pltpu.SemaphoreType.DMA((HOPS,)),
            pltpu.SemaphoreType.DMA((HOPS,)),
            pltpu.SemaphoreType.DMA((HOPS,)),
            pltpu.SemaphoreType.DMA((HOPS,)),
        ],
        compiler_params=pltpu.CompilerParams(collective_id=0),
    )(meta, tbl, x)


_RB = 256


def _layer_body(x_ref, win_ref, wout_ref, out_ref):
    h = jnp.maximum(
        jnp.dot(x_ref[...], win_ref[...], preferred_element_type=jnp.float32),
        0.0,
    )
    out_ref[...] = jnp.dot(h, wout_ref[...], preferred_element_type=jnp.float32)


def _layer(x_full, win, wout):
    hdim = win.shape[1]
    return pl.pallas_call(
        _layer_body,
        grid=(B // _RB,),
        in_specs=[
            pl.BlockSpec((_RB, D), lambda r: (r, 0)),
            pl.BlockSpec((D, hdim), lambda r: (0, 0)),
            pl.BlockSpec((hdim, D), lambda r: (0, 0)),
        ],
        out_specs=pl.BlockSpec((_RB, D), lambda r: (r, 0)),
        out_shape=jax.ShapeDtypeStruct((B, D), jnp.float32),
    )(x_full, win, wout)


def _ar_body(meta_ref, p_ref, out_ref, ct, cb,
             t_rs_s, t_rs_r, t_ag_s, t_ag_r,
             b_rs_s, b_rs_r, b_ag_s, b_ag_r):
    pos, nxt, prv, _ = (meta_ref[i] for i in range(4))
    _neighbor_barrier(nxt, prv)

    def toprows(c):
        return pl.ds(c * C2, C2)

    def botrows(c):
        return pl.ds(HALF + c * C2, C2)

    ct[0, :, :] = p_ref[toprows(pos), :]
    cb[0, :, :] = p_ref[botrows(pos), :]

    def acc_t(h, rr):
        tc = (pos - h - 1) % N_DEV
        ct[rr, :, :] = ct[rr] + p_ref[toprows(tc), :]

    def acc_b(h, rr):
        bc = (pos + h + 1) % N_DEV
        cb[rr, :, :] = cb[rr] + p_ref[botrows(bc), :]

    _dual_ring(ct, cb, t_rs_s, t_rs_r, b_rs_s, b_rs_r, nxt, prv, 0,
               acc_t, acc_b, mutates=True)

    last = HOPS % NSLOT
    out_ref[toprows((pos + 1) % N_DEV), :] = ct[last]
    out_ref[botrows((pos - 1) % N_DEV), :] = cb[last]

    def store_t(h, rr):
        out_ref[toprows((pos - h) % N_DEV), :] = ct[rr]

    def store_b(h, rr):
        out_ref[botrows((pos + h) % N_DEV), :] = cb[rr]

    _dual_ring(ct, cb, t_ag_s, t_ag_r, b_ag_s, b_ag_r, nxt, prv, last,
               store_t, store_b)


def _allreduce(p, meta):
    return pl.pallas_call(
        _ar_body,
        out_shape=jax.ShapeDtypeStruct((B, D), p.dtype),
        in_specs=[
            pl.BlockSpec(memory_space=pltpu.SMEM),
            pl.BlockSpec(memory_space=pltpu.VMEM),
        ],
        out_specs=pl.BlockSpec(memory_space=pltpu.VMEM),
        scratch_shapes=[
            pltpu.VMEM((NSLOT, C2, D), p.dtype),
            pltpu.VMEM((NSLOT, C2, D), p.dtype),
        ] + [pltpu.SemaphoreType.DMA((HOPS,)) for _ in range(8)],
        compiler_params=pltpu.CompilerParams(collective_id=1),
    )(meta, p)


def kernel(x, Win0, Wout0, Win1, Wout1, Win2, Wout2):
    meta = _ring_meta()
    tbl = jnp.asarray(_CYC_LID)
    x_full = _allgather(x, meta, tbl)
    for win, wout in ((Win0, Wout0), (Win1, Wout1), (Win2, Wout2)):
        partial = _layer(x_full, win, wout)
        x_full = _allreduce(partial, meta)
    return x_full


# device time: 602504 ns/iter; 2.1455x vs baseline; 1.2994x over previous
import numpy as np

import jax
import jax.numpy as jnp
from jax import lax
from jax.experimental import pallas as pl
from jax.experimental.pallas import tpu as pltpu

N_DEV = 32
B = 2048
D = 1024
CH = B // N_DEV
C2 = CH // 2
HALF = B // 2
DH = D // 2
NSLOT = 4
HOPS = N_DEV - 1
SERIALIZED = False


def _build_cycle():
    logical_coords = []
    for z in range(4):
        for y in range(4):
            row = [(0, y, z), (1, y, z)] if y % 2 == 0 else [(1, y, z), (0, y, z)]
            logical_coords.extend(row)
    lid = {c: i for i, c in enumerate(logical_coords)}
    yz = []
    for z in range(4):
        ys = range(4) if z % 2 == 0 else range(3, -1, -1)
        yz.extend((y, z) for y in ys)
    cycle = [(0, y, z) for (y, z) in yz] + [(1, y, z) for (y, z) in reversed(yz)]
    cyc_lid = [lid[c] for c in cycle]
    pos = np.argsort(cyc_lid).astype(np.int32)
    nxt = np.empty(N_DEV, np.int32)
    prv = np.empty(N_DEV, np.int32)
    for p, l in enumerate(cyc_lid):
        nxt[l] = cyc_lid[(p + 1) % N_DEV]
        prv[l] = cyc_lid[(p - 1) % N_DEV]

    plane_cycle = [(0, 0), (0, 1), (0, 2), (0, 3), (1, 3), (1, 2), (1, 1), (1, 0)]
    ppos = np.empty(N_DEV, np.int32)
    pnxt = np.empty(N_DEV, np.int32)
    pprv = np.empty(N_DEV, np.int32)
    zpos = np.empty(N_DEV, np.int32)
    znxt = np.empty(N_DEV, np.int32)
    zprv = np.empty(N_DEV, np.int32)
    for (x, y, z), l in lid.items():
        k = plane_cycle.index((x, y))
        ppos[l] = k
        pnxt[l] = lid[plane_cycle[(k + 1) % 8] + (z,)]
        pprv[l] = lid[plane_cycle[(k - 1) % 8] + (z,)]
        zpos[l] = z
        znxt[l] = lid[(x, y, (z + 1) % 4)]
        zprv[l] = lid[(x, y, (z - 1) % 4)]
    return (np.array(cyc_lid, np.int32), pos, nxt, prv,
            ppos, pnxt, pprv, zpos, znxt, zprv)


(_CYC_LID, _POS, _NXT, _PRV,
 _PPOS, _PNXT, _PPRV, _ZPOS, _ZNXT, _ZPRV) = _build_cycle()


def _ring_meta():
    my = lax.axis_index("i")
    vals = [jnp.asarray(t)[my] for t in
            (_POS, _NXT, _PRV)] + [my] + [jnp.asarray(t)[my] for t in
            (_PPOS, _PNXT, _PPRV, _ZPOS, _ZNXT, _ZPRV)]
    return jnp.stack(vals).astype(jnp.int32)


def _neighbor_barrier(*nbrs):
    barrier = pltpu.get_barrier_semaphore()
    for nbr in nbrs:
        pl.semaphore_signal(
            barrier, inc=1, device_id=(nbr,),
            device_id_type=pl.DeviceIdType.MESH,
        )
    pl.semaphore_wait(barrier, len(nbrs))


def _rdma(src, dst, send_sem, recv_sem, dev):
    return pltpu.make_async_remote_copy(
        src_ref=src, dst_ref=dst, send_sem=send_sem, recv_sem=recv_sem,
        device_id=(dev,), device_id_type=pl.DeviceIdType.MESH,
    )


def _dual_ring(ct, cb, t_s, t_r, b_s, b_r, nxt, prv, s0, on_t, on_b,
               mutates=False, hops=HOPS):
    if SERIALIZED:
        for h in range(hops):
            ss, rr = (s0 + h) % NSLOT, (s0 + h + 1) % NSLOT
            rt = _rdma(ct.at[ss], ct.at[rr], t_s.at[h], t_r.at[h], nxt)
            rb = _rdma(cb.at[ss], cb.at[rr], b_s.at[h], b_r.at[h], prv)
            rt.start()
            rb.start()
            rt.wait()
            rb.wait()
            on_t(h, rr)
            on_b(h, rr)
        return
    rts, rbs = [], []
    for h in range(hops):
        ss, rr = (s0 + h) % NSLOT, (s0 + h + 1) % NSLOT
        rts.append(_rdma(ct.at[ss], ct.at[rr], t_s.at[h], t_r.at[h], nxt))
        rbs.append(_rdma(cb.at[ss], cb.at[rr], b_s.at[h], b_r.at[h], prv))
    rts[0].start()
    rbs[0].start()
    for h in range(hops):
        rr = (s0 + h + 1) % NSLOT
        rts[h].wait_recv()
        if mutates:
            on_t(h, rr)
        if h + 1 < hops:
            rts[h + 1].start()
        rbs[h].wait_recv()
        if mutates:
            on_b(h, rr)
        if h + 1 < hops:
            rbs[h + 1].start()
        if not mutates:
            on_t(h, rr)
            on_b(h, rr)
        if h >= 2:
            rts[h - 2].wait_send()
            rbs[h - 2].wait_send()
    for r in rts[-2:] + rbs[-2:]:
        r.wait_send()


def _ag_body(meta_ref, tbl_ref, x_ref, out_ref, ct, cb, t_s, t_r, b_s, b_r):
    pos, nxt, prv, my = (meta_ref[i] for i in range(4))
    _neighbor_barrier(nxt, prv)

    out_ref[pl.ds(my * CH, CH), :] = x_ref[...]
    ct[0, :, :] = x_ref[:, 0:DH]
    cb[0, :, :] = x_ref[:, DH:D]

    def store_t(h, rr):
        to = tbl_ref[(pos - h - 1) % N_DEV]
        out_ref[pl.ds(to * CH, CH), 0:DH] = ct[rr]

    def store_b(h, rr):
        bo = tbl_ref[(pos + h + 1) % N_DEV]
        out_ref[pl.ds(bo * CH, CH), DH:D] = cb[rr]

    _dual_ring(ct, cb, t_s, t_r, b_s, b_r, nxt, prv, 0, store_t, store_b)


def _allgather(x, meta, tbl):
    return pl.pallas_call(
        _ag_body,
        out_shape=jax.ShapeDtypeStruct((B, D), x.dtype),
        in_specs=[
            pl.BlockSpec(memory_space=pltpu.SMEM),
            pl.BlockSpec(memory_space=pltpu.SMEM),
            pl.BlockSpec(memory_space=pltpu.VMEM),
        ],
        out_specs=pl.BlockSpec(memory_space=pltpu.VMEM),
        scratch_shapes=[
            pltpu.VMEM((NSLOT, CH, DH), x.dtype),
            pltpu.VMEM((NSLOT, CH, DH), x.dtype),
            pltpu.SemaphoreType.DMA((HOPS,)),
            pltpu.SemaphoreType.DMA((HOPS,)),
            pltpu.SemaphoreType.DMA((HOPS,)),
            pltpu.SemaphoreType.DMA((HOPS,)),
        ],
        compiler_params=pltpu.CompilerParams(collective_id=0),
    )(meta, tbl, x)


_RB = 256


def _layer_body(x_ref, win_ref, wout_ref, out_ref):
    h = jnp.maximum(
        jnp.dot(x_ref[...], win_ref[...], preferred_element_type=jnp.float32),
        0.0,
    )
    out_ref[...] = jnp.dot(h, wout_ref[...], preferred_element_type=jnp.float32)


def _layer(x_full, win, wout):
    hdim = win.shape[1]
    return pl.pallas_call(
        _layer_body,
        grid=(B // _RB,),
        in_specs=[
            pl.BlockSpec((_RB, D), lambda r: (r, 0)),
            pl.BlockSpec((D, hdim), lambda r: (0, 0)),
            pl.BlockSpec((hdim, D), lambda r: (0, 0)),
        ],
        out_specs=pl.BlockSpec((_RB, D), lambda r: (r, 0)),
        out_shape=jax.ShapeDtypeStruct((B, D), jnp.float32),
    )(x_full, win, wout)


P_HOPS = 7
Z_HOPS = 3
SUP = HALF // 8
SUB = SUP // 4


def _ar_body(meta_ref, p_ref, out_ref, ct, cb, zt, zb, gt, gb,
             p1ts, p1tr, p1bs, p1br,
             p2ts, p2tr, p2bs, p2br,
             p3ts, p3tr, p3bs, p3br,
             p4ts, p4tr, p4bs, p4br):
    ppos, pnxt, pprv, zpos, znxt, zprv = (meta_ref[i] for i in range(4, 10))
    _neighbor_barrier(pnxt, pprv, znxt, zprv)

    def tsup(c):
        return pl.ds(c * SUP, SUP)

    def bsup(c):
        return pl.ds(HALF + c * SUP, SUP)

    ct[0, :, :] = p_ref[tsup(ppos), :]
    cb[0, :, :] = p_ref[bsup(ppos), :]

    def acc_t(h, rr):
        c = (ppos - h - 1) % 8
        ct[rr, :, :] = ct[rr] + p_ref[tsup(c), :]

    def acc_b(h, rr):
        c = (ppos + h + 1) % 8
        cb[rr, :, :] = cb[rr] + p_ref[bsup(c), :]

    _dual_ring(ct, cb, p1ts, p1tr, p1bs, p1br, pnxt, pprv, 0,
               acc_t, acc_b, mutates=True, hops=P_HOPS)
    l1 = P_HOPS % NSLOT

    zt[0, :, :] = ct[l1, pl.ds(zpos * SUB, SUB), :]
    zb[0, :, :] = cb[l1, pl.ds(zpos * SUB, SUB), :]

    def zacc_t(h, rr):
        j = (zpos - h - 1) % 4
        zt[rr, :, :] = zt[rr] + ct[l1, pl.ds(j * SUB, SUB), :]

    def zacc_b(h, rr):
        j = (zpos + h + 1) % 4
        zb[rr, :, :] = zb[rr] + cb[l1, pl.ds(j * SUB, SUB), :]

    _dual_ring(zt, zb, p2ts, p2tr, p2bs, p2br, znxt, zprv, 0,
               zacc_t, zacc_b, mutates=True, hops=Z_HOPS)
    l2 = Z_HOPS % NSLOT

    st = (ppos + 1) % 8
    sb = (ppos - 1) % 8
    jt = (zpos + 1) % 4
    jb = (zpos - 1) % 4
    gt[0, pl.ds(jt * SUB, SUB), :] = zt[l2]
    gb[0, pl.ds(jb * SUB, SUB), :] = zb[l2]
    out_ref[pl.ds(st * SUP + jt * SUB, SUB), :] = zt[l2]
    out_ref[pl.ds(HALF + sb * SUP + jb * SUB, SUB), :] = zb[l2]

    def zst_t(h, rr):
        j = (zpos - h) % 4
        gt[0, pl.ds(j * SUB, SUB), :] = zt[rr]
        out_ref[pl.ds(st * SUP + j * SUB, SUB), :] = zt[rr]

    def zst_b(h, rr):
        j = (zpos + h) % 4
        gb[0, pl.ds(j * SUB, SUB), :] = zb[rr]
        out_ref[pl.ds(HALF + sb * SUP + j * SUB, SUB), :] = zb[rr]

    _dual_ring(zt, zb, p3ts, p3tr, p3bs, p3br, znxt, zprv, l2,
               zst_t, zst_b, hops=Z_HOPS)

    def gst_t(h, rr):
        c = (ppos - h) % 8
        out_ref[tsup(c), :] = gt[rr]

    def gst_b(h, rr):
        c = (ppos + h) % 8
        out_ref[bsup(c), :] = gb[rr]

    _dual_ring(gt, gb, p4ts, p4tr, p4bs, p4br, pnxt, pprv, 0,
               gst_t, gst_b, hops=P_HOPS)


def _allreduce(p, meta):
    sems = ([pltpu.SemaphoreType.DMA((P_HOPS,)) for _ in range(4)]
            + [pltpu.SemaphoreType.DMA((Z_HOPS,)) for _ in range(8)]
            + [pltpu.SemaphoreType.DMA((P_HOPS,)) for _ in range(4)])
    return pl.pallas_call(
        _ar_body,
        out_shape=jax.ShapeDtypeStruct((B, D), p.dtype),
        in_specs=[
            pl.BlockSpec(memory_space=pltpu.SMEM),
            pl.BlockSpec(memory_space=pltpu.VMEM),
        ],
        out_specs=pl.BlockSpec(memory_space=pltpu.VMEM),
        scratch_shapes=[
            pltpu.VMEM((NSLOT, SUP, D), p.dtype),
            pltpu.VMEM((NSLOT, SUP, D), p.dtype),
            pltpu.VMEM((NSLOT, SUB, D), p.dtype),
            pltpu.VMEM((NSLOT, SUB, D), p.dtype),
            pltpu.VMEM((NSLOT, SUP, D), p.dtype),
            pltpu.VMEM((NSLOT, SUP, D), p.dtype),
        ] + sems,
        compiler_params=pltpu.CompilerParams(collective_id=1),
    )(meta, p)


def kernel(x, Win0, Wout0, Win1, Wout1, Win2, Wout2):
    meta = _ring_meta()
    tbl = jnp.asarray(_CYC_LID)
    x_full = _allgather(x, meta, tbl)
    for win, wout in ((Win0, Wout0), (Win1, Wout1), (Win2, Wout2)):
        partial = _layer(x_full, win, wout)
        x_full = _allreduce(partial, meta)
    return x_full


# device time: 572033 ns/iter; 2.2598x vs baseline; 1.0533x over previous
import numpy as np

import jax
import jax.numpy as jnp
from jax import lax
from jax.experimental import pallas as pl
from jax.experimental.pallas import tpu as pltpu

N_DEV = 32
B = 2048
D = 1024
CH = B // N_DEV
C2 = CH // 2
HALF = B // 2
DH = D // 2
NSLOT = 4
HOPS = N_DEV - 1
SERIALIZED = False


def _build_cycle():
    logical_coords = []
    for z in range(4):
        for y in range(4):
            row = [(0, y, z), (1, y, z)] if y % 2 == 0 else [(1, y, z), (0, y, z)]
            logical_coords.extend(row)
    lid = {c: i for i, c in enumerate(logical_coords)}
    yz = []
    for z in range(4):
        ys = range(4) if z % 2 == 0 else range(3, -1, -1)
        yz.extend((y, z) for y in ys)
    cycle = [(0, y, z) for (y, z) in yz] + [(1, y, z) for (y, z) in reversed(yz)]
    cyc_lid = [lid[c] for c in cycle]
    pos = np.argsort(cyc_lid).astype(np.int32)
    nxt = np.empty(N_DEV, np.int32)
    prv = np.empty(N_DEV, np.int32)
    for p, l in enumerate(cyc_lid):
        nxt[l] = cyc_lid[(p + 1) % N_DEV]
        prv[l] = cyc_lid[(p - 1) % N_DEV]

    plane_cycle = [(0, 0), (0, 1), (0, 2), (0, 3), (1, 3), (1, 2), (1, 1), (1, 0)]
    ppos = np.empty(N_DEV, np.int32)
    pnxt = np.empty(N_DEV, np.int32)
    pprv = np.empty(N_DEV, np.int32)
    zpos = np.empty(N_DEV, np.int32)
    znxt = np.empty(N_DEV, np.int32)
    zprv = np.empty(N_DEV, np.int32)
    pln = np.empty((N_DEV, 8), np.int32)
    col = np.empty((N_DEV, 4), np.int32)
    for (x, y, z), l in lid.items():
        k = plane_cycle.index((x, y))
        ppos[l] = k
        pnxt[l] = lid[plane_cycle[(k + 1) % 8] + (z,)]
        pprv[l] = lid[plane_cycle[(k - 1) % 8] + (z,)]
        zpos[l] = z
        znxt[l] = lid[(x, y, (z + 1) % 4)]
        zprv[l] = lid[(x, y, (z - 1) % 4)]
        pln[l] = [lid[pc + (z,)] for pc in plane_cycle]
        col[l] = [lid[(x, y, j)] for j in range(4)]
    return (np.array(cyc_lid, np.int32), pos, nxt, prv,
            ppos, pnxt, pprv, zpos, znxt, zprv, pln, col)


(_CYC_LID, _POS, _NXT, _PRV,
 _PPOS, _PNXT, _PPRV, _ZPOS, _ZNXT, _ZPRV, _PLN, _COL) = _build_cycle()


def _ring_meta():
    my = lax.axis_index("i")
    vals = [jnp.asarray(t)[my] for t in
            (_POS, _NXT, _PRV)] + [my] + [jnp.asarray(t)[my] for t in
            (_PPOS, _PNXT, _PPRV, _ZPOS, _ZNXT, _ZPRV)]
    return jnp.stack(vals).astype(jnp.int32)


def _neighbor_barrier(*nbrs):
    barrier = pltpu.get_barrier_semaphore()
    for nbr in nbrs:
        pl.semaphore_signal(
            barrier, inc=1, device_id=(nbr,),
            device_id_type=pl.DeviceIdType.MESH,
        )
    pl.semaphore_wait(barrier, len(nbrs))


def _rdma(src, dst, send_sem, recv_sem, dev):
    return pltpu.make_async_remote_copy(
        src_ref=src, dst_ref=dst, send_sem=send_sem, recv_sem=recv_sem,
        device_id=(dev,), device_id_type=pl.DeviceIdType.MESH,
    )


def _dual_ring(ct, cb, t_s, t_r, b_s, b_r, nxt, prv, s0, on_t, on_b,
               mutates=False, hops=HOPS):
    if SERIALIZED:
        for h in range(hops):
            ss, rr = (s0 + h) % NSLOT, (s0 + h + 1) % NSLOT
            rt = _rdma(ct.at[ss], ct.at[rr], t_s.at[h], t_r.at[h], nxt)
            rb = _rdma(cb.at[ss], cb.at[rr], b_s.at[h], b_r.at[h], prv)
            rt.start()
            rb.start()
            rt.wait()
            rb.wait()
            on_t(h, rr)
            on_b(h, rr)
        return
    rts, rbs = [], []
    for h in range(hops):
        ss, rr = (s0 + h) % NSLOT, (s0 + h + 1) % NSLOT
        rts.append(_rdma(ct.at[ss], ct.at[rr], t_s.at[h], t_r.at[h], nxt))
        rbs.append(_rdma(cb.at[ss], cb.at[rr], b_s.at[h], b_r.at[h], prv))
    rts[0].start()
    rbs[0].start()
    for h in range(hops):
        rr = (s0 + h + 1) % NSLOT
        rts[h].wait_recv()
        if mutates:
            on_t(h, rr)
        if h + 1 < hops:
            rts[h + 1].start()
        rbs[h].wait_recv()
        if mutates:
            on_b(h, rr)
        if h + 1 < hops:
            rbs[h + 1].start()
        if not mutates:
            on_t(h, rr)
            on_b(h, rr)
        if h >= 2:
            rts[h - 2].wait_send()
            rbs[h - 2].wait_send()
    for r in rts[-2:] + rbs[-2:]:
        r.wait_send()


def _ag_body(meta_ref, col_ref, pln_ref, x_ref, out_ref, za, zb, ga, gb,
             z_ts, z_tr, z_bs, z_br, p_ts, p_tr, p_bs, p_br):
    myl = meta_ref[3]
    ppos, pnxt, pprv, zpos, znxt, zprv = (meta_ref[i] for i in range(4, 10))
    _neighbor_barrier(pnxt, pprv, znxt, zprv)

    out_ref[pl.ds(myl * CH, CH), :] = x_ref[...]
    ga[0, pl.ds(zpos * CH, CH), :] = x_ref[:, 0:DH]
    gb[0, pl.ds(zpos * CH, CH), :] = x_ref[:, DH:D]
    za[0, :, :] = x_ref[:, 0:DH]
    zb[0, :, :] = x_ref[:, DH:D]

    def zst_t(h, rr):
        j = (zpos - h - 1) % 4
        ga[0, pl.ds(j * CH, CH), :] = za[rr]
        out_ref[pl.ds(col_ref[myl, j] * CH, CH), 0:DH] = za[rr]

    def zst_b(h, rr):
        j = (zpos + h + 1) % 4
        gb[0, pl.ds(j * CH, CH), :] = zb[rr]
        out_ref[pl.ds(col_ref[myl, j] * CH, CH), DH:D] = zb[rr]

    _dual_ring(za, zb, z_ts, z_tr, z_bs, z_br, znxt, zprv, 0,
               zst_t, zst_b, hops=Z_HOPS)

    def pst_t(h, rr):
        pm = pln_ref[(ppos - h - 1) % 8]
        for j in range(4):
            out_ref[pl.ds(col_ref[pm, j] * CH, CH), 0:DH] = \
                ga[rr, pl.ds(j * CH, CH), :]

    def pst_b(h, rr):
        pm = pln_ref[(ppos + h + 1) % 8]
        for j in range(4):
            out_ref[pl.ds(col_ref[pm, j] * CH, CH), DH:D] = \
                gb[rr, pl.ds(j * CH, CH), :]

    _dual_ring(ga, gb, p_ts, p_tr, p_bs, p_br, pnxt, pprv, 0,
               pst_t, pst_b, hops=P_HOPS)


def _allgather(x, meta, col2, pln8):
    return pl.pallas_call(
        _ag_body,
        out_shape=jax.ShapeDtypeStruct((B, D), x.dtype),
        in_specs=[
            pl.BlockSpec(memory_space=pltpu.SMEM),
            pl.BlockSpec(memory_space=pltpu.SMEM),
            pl.BlockSpec(memory_space=pltpu.SMEM),
            pl.BlockSpec(memory_space=pltpu.VMEM),
        ],
        out_specs=pl.BlockSpec(memory_space=pltpu.VMEM),
        scratch_shapes=[
            pltpu.VMEM((NSLOT, CH, DH), x.dtype),
            pltpu.VMEM((NSLOT, CH, DH), x.dtype),
            pltpu.VMEM((NSLOT, 4 * CH, DH), x.dtype),
            pltpu.VMEM((NSLOT, 4 * CH, DH), x.dtype),
        ] + [pltpu.SemaphoreType.DMA((Z_HOPS,)) for _ in range(4)]
          + [pltpu.SemaphoreType.DMA((P_HOPS,)) for _ in range(4)],
        compiler_params=pltpu.CompilerParams(collective_id=0),
    )(meta, col2, pln8, x)


_RB = 256


def _layer_body(x_ref, win_ref, wout_ref, out_ref):
    h = jnp.maximum(
        jnp.dot(x_ref[...], win_ref[...], preferred_element_type=jnp.float32),
        0.0,
    )
    out_ref[...] = jnp.dot(h, wout_ref[...], preferred_element_type=jnp.float32)


def _layer(x_full, win, wout):
    hdim = win.shape[1]
    return pl.pallas_call(
        _layer_body,
        grid=(B // _RB,),
        in_specs=[
            pl.BlockSpec((_RB, D), lambda r: (r, 0)),
            pl.BlockSpec((D, hdim), lambda r: (0, 0)),
            pl.BlockSpec((hdim, D), lambda r: (0, 0)),
        ],
        out_specs=pl.BlockSpec((_RB, D), lambda r: (r, 0)),
        out_shape=jax.ShapeDtypeStruct((B, D), jnp.float32),
    )(x_full, win, wout)


P_HOPS = 7
Z_HOPS = 3
SUP = HALF // 8
SUB = SUP // 4


def _ar_body(meta_ref, p_ref, out_ref, ct, cb, zt, zb, gt, gb,
             p1ts, p1tr, p1bs, p1br,
             p2ts, p2tr, p2bs, p2br,
             p3ts, p3tr, p3bs, p3br,
             p4ts, p4tr, p4bs, p4br):
    ppos, pnxt, pprv, zpos, znxt, zprv = (meta_ref[i] for i in range(4, 10))
    _neighbor_barrier(pnxt, pprv, znxt, zprv)

    def tsup(c):
        return pl.ds(c * SUP, SUP)

    def bsup(c):
        return pl.ds(HALF + c * SUP, SUP)

    ct[0, :, :] = p_ref[tsup(ppos), :]
    cb[0, :, :] = p_ref[bsup(ppos), :]

    def acc_t(h, rr):
        c = (ppos - h - 1) % 8
        ct[rr, :, :] = ct[rr] + p_ref[tsup(c), :]

    def acc_b(h, rr):
        c = (ppos + h + 1) % 8
        cb[rr, :, :] = cb[rr] + p_ref[bsup(c), :]

    _dual_ring(ct, cb, p1ts, p1tr, p1bs, p1br, pnxt, pprv, 0,
               acc_t, acc_b, mutates=True, hops=P_HOPS)
    l1 = P_HOPS % NSLOT

    zt[0, :, :] = ct[l1, pl.ds(zpos * SUB, SUB), :]
    zb[0, :, :] = cb[l1, pl.ds(zpos * SUB, SUB), :]

    def zacc_t(h, rr):
        j = (zpos - h - 1) % 4
        zt[rr, :, :] = zt[rr] + ct[l1, pl.ds(j * SUB, SUB), :]

    def zacc_b(h, rr):
        j = (zpos + h + 1) % 4
        zb[rr, :, :] = zb[rr] + cb[l1, pl.ds(j * SUB, SUB), :]

    _dual_ring(zt, zb, p2ts, p2tr, p2bs, p2br, znxt, zprv, 0,
               zacc_t, zacc_b, mutates=True, hops=Z_HOPS)
    l2 = Z_HOPS % NSLOT

    st = (ppos + 1) % 8
    sb = (ppos - 1) % 8
    jt = (zpos + 1) % 4
    jb = (zpos - 1) % 4
    gt[0, pl.ds(jt * SUB, SUB), :] = zt[l2]
    gb[0, pl.ds(jb * SUB, SUB), :] = zb[l2]
    out_ref[pl.ds(st * SUP + jt * SUB, SUB), :] = zt[l2]
    out_ref[pl.ds(HALF + sb * SUP + jb * SUB, SUB), :] = zb[l2]

    def zst_t(h, rr):
        j = (zpos - h) % 4
        gt[0, pl.ds(j * SUB, SUB), :] = zt[rr]
        out_ref[pl.ds(st * SUP + j * SUB, SUB), :] = zt[rr]

    def zst_b(h, rr):
        j = (zpos + h) % 4
        gb[0, pl.ds(j * SUB, SUB), :] = zb[rr]
        out_ref[pl.ds(HALF + sb * SUP + j * SUB, SUB), :] = zb[rr]

    _dual_ring(zt, zb, p3ts, p3tr, p3bs, p3br, znxt, zprv, l2,
               zst_t, zst_b, hops=Z_HOPS)

    def gst_t(h, rr):
        c = (ppos - h) % 8
        out_ref[tsup(c), :] = gt[rr]

    def gst_b(h, rr):
        c = (ppos + h) % 8
        out_ref[bsup(c), :] = gb[rr]

    _dual_ring(gt, gb, p4ts, p4tr, p4bs, p4br, pnxt, pprv, 0,
               gst_t, gst_b, hops=P_HOPS)


def _allreduce(p, meta):
    sems = ([pltpu.SemaphoreType.DMA((P_HOPS,)) for _ in range(4)]
            + [pltpu.SemaphoreType.DMA((Z_HOPS,)) for _ in range(8)]
            + [pltpu.SemaphoreType.DMA((P_HOPS,)) for _ in range(4)])
    return pl.pallas_call(
        _ar_body,
        out_shape=jax.ShapeDtypeStruct((B, D), p.dtype),
        in_specs=[
            pl.BlockSpec(memory_space=pltpu.SMEM),
            pl.BlockSpec(memory_space=pltpu.VMEM),
        ],
        out_specs=pl.BlockSpec(memory_space=pltpu.VMEM),
        scratch_shapes=[
            pltpu.VMEM((NSLOT, SUP, D), p.dtype),
            pltpu.VMEM((NSLOT, SUP, D), p.dtype),
            pltpu.VMEM((NSLOT, SUB, D), p.dtype),
            pltpu.VMEM((NSLOT, SUB, D), p.dtype),
            pltpu.VMEM((NSLOT, SUP, D), p.dtype),
            pltpu.VMEM((NSLOT, SUP, D), p.dtype),
        ] + sems,
        compiler_params=pltpu.CompilerParams(collective_id=1),
    )(meta, p)


def kernel(x, Win0, Wout0, Win1, Wout1, Win2, Wout2):
    meta = _ring_meta()
    my = lax.axis_index("i")
    col2 = jnp.asarray(_COL)
    pln8 = jnp.asarray(_PLN)[my]
    x_full = _allgather(x, meta, col2, pln8)
    for win, wout in ((Win0, Wout0), (Win1, Wout1), (Win2, Wout2)):
        partial = _layer(x_full, win, wout)
        x_full = _allreduce(partial, meta)
    return x_full
